# Initial kernel scaffold; baseline (speedup 1.0000x reference)
import jax
import jax.numpy as jnp
from jax.experimental import pallas as pl


def kernel(proj_points, threshold):
    raise NotImplementedError("write your pallas kernel here")


# speedup vs baseline: 78.1081x; 78.1081x over previous
import functools

import jax
import jax.numpy as jnp
from jax import lax
from jax.experimental import pallas as pl
from jax.experimental.pallas import tpu as pltpu
from jax.experimental.pallas import tpu_sc as plsc

H, W, P, EPS = 512, 512, 5, 1e-05
N = 262144
NC, NS, L = 2, 16, 16
NW = NC * NS
PPW = N // NW
CH = 1024
NCHUNK = PPW // CH
KPP = P * P
SLICE = H * W // NS
HALF = (P - 1) / 2.0


def _sc_body(xs_hbm, ys_hbm, out_hbm, xs_v, ys_v, vals_v, idx_v, acc_sh):
    c = lax.axis_index("c")
    s = lax.axis_index("s")
    wid = c * NS + s
    base = wid * PPW

    zero16 = jnp.zeros((L,), jnp.float32)

    def _zero(i, carry):
        vals_v[pl.ds(i * L, L)] = zero16
        return carry

    lax.fori_loop(0, SLICE // L, _zero, 0)
    pltpu.sync_copy(vals_v.at[pl.ds(0, SLICE)], acc_sh.at[pl.ds(s * SLICE, SLICE)])

    pltpu.sync_copy(xs_hbm.at[pl.ds(base, PPW)], xs_v)
    pltpu.sync_copy(ys_hbm.at[pl.ds(base, PPW)], ys_v)
    plsc.subcore_barrier()

    for ci in range(NCHUNK):
        def _compute(g, carry, ci=ci):
            off = ci * CH + g * L
            x = xs_v[pl.ds(off, L)]
            y = ys_v[pl.ds(off, L)]
            tx = x.astype(jnp.int32)
            ty = y.astype(jnp.int32)
            fx = x - tx.astype(jnp.float32)
            fy = y - ty.astype(jnp.float32)
            bx = tx + jnp.where(fx >= 0.5, 1, 0)
            by = ty + jnp.where(fy >= 0.5, 1, 0)
            ddx = bx.astype(jnp.float32) - x
            ddy = by.astype(jnp.float32) - y
            wxs, cols = [], []
            for i in range(P):
                o = i - P // 2
                pxi = bx + o
                wx = jnp.clip(HALF + 0.5 - jnp.abs(ddx + float(o)), 0.0, 1.0)
                vx = (pxi >= 0) & (pxi < W)
                wxs.append(jnp.where(vx, wx, 0.0))
                cols.append(jnp.clip(pxi, 0, W - 1))
            for j in range(P):
                o = j - P // 2
                pyj = by + o
                wy = jnp.clip(HALF + 0.5 - jnp.abs(ddy + float(o)), 0.0, 1.0)
                vy = (pyj >= 0) & (pyj < H)
                wyv = jnp.where(vy, wy, 0.0)
                rowb = jnp.clip(pyj, 0, H - 1) * W
                for i in range(P):
                    pos = g * (KPP * L) + (j * P + i) * L
                    vals_v[pl.ds(pos, L)] = wyv * wxs[i]
                    idx_v[pl.ds(pos, L)] = rowb + cols[i]
            return carry

        lax.fori_loop(0, CH // L, _compute, 0)
        pltpu.sync_copy(vals_v, acc_sh.at[idx_v], add=True)

    plsc.subcore_barrier()
    pltpu.sync_copy(acc_sh.at[pl.ds(s * SLICE, SLICE)], vals_v.at[pl.ds(0, SLICE)])
    pltpu.sync_copy(vals_v.at[pl.ds(0, SLICE)], out_hbm.at[c, pl.ds(s * SLICE, SLICE)])


_sc_render = functools.partial(
    pl.kernel,
    out_type=jax.ShapeDtypeStruct((NC, H * W), jnp.float32),
    mesh=plsc.VectorSubcoreMesh(core_axis_name="c", subcore_axis_name="s"),
    scratch_types=[
        pltpu.VMEM((PPW,), jnp.float32),
        pltpu.VMEM((PPW,), jnp.float32),
        pltpu.VMEM((CH * KPP,), jnp.float32),
        pltpu.VMEM((CH * KPP,), jnp.int32),
        pltpu.VMEM_SHARED((H * W,), jnp.float32),
    ],
)(_sc_body)


def _combine_body(thr_ref, p_ref, o_ref):
    thr = thr_ref[0]
    o_ref[:, :] = jnp.clip((p_ref[0] + p_ref[1]) / thr, 0.0, 1.0)


def _combine(partials, thr):
    return pl.pallas_call(
        _combine_body,
        out_shape=jax.ShapeDtypeStruct((H, W), jnp.float32),
        in_specs=[
            pl.BlockSpec(memory_space=pltpu.SMEM),
            pl.BlockSpec(memory_space=pltpu.VMEM),
        ],
        out_specs=pl.BlockSpec(memory_space=pltpu.VMEM),
    )(thr, partials)


def kernel(proj_points, threshold):
    xs = proj_points[:, 0]
    ys = proj_points[:, 1]
    partials = _sc_render(xs, ys).reshape(NC, H, W)
    thr = jnp.maximum(jnp.asarray(threshold, jnp.float32), EPS).reshape(1)
    return _combine(partials, thr)
